# baseline (speedup 1.0000x reference)
import jax
import jax.numpy as jnp
from jax.experimental import pallas as pl
from jax.experimental.pallas import tpu as pltpu


def _stage_a_kernel(x1_ref, x2_ref, w1_ref, w2_ref, b1_ref, b2_ref, y_ref, s_ref):
    T = x1_ref.shape[1]
    bf16 = jnp.bfloat16
    x1 = [x1_ref[0, t] for t in range(T)]
    x2 = [x2_ref[0, t] for t in range(T)]
    qT = b1_ref[...] + sum(
        jax.lax.dot_general(x1[t], w1_ref[0, t], (((1,), (1,)), ((), ())),
                            preferred_element_type=jnp.float32)
        for t in range(T))
    kT = b2_ref[...] + sum(
        jax.lax.dot_general(x2[t], w2_ref[0, t], (((1,), (1,)), ((), ())),
                            preferred_element_type=jnp.float32)
        for t in range(T))
    logits = jax.lax.dot_general(
        qT, kT, (((1,), (1,)), ((), ())), preferred_element_type=jnp.float32)
    m = jnp.max(logits, axis=0, keepdims=True)
    e = jnp.exp(logits - m)
    p = e / jnp.sum(e, axis=0, keepdims=True)
    s_ref[0] = jnp.sum(p, axis=1, keepdims=True)
    pb = p.astype(bf16)
    for t in range(T):
        y_ref[0, t] = jax.lax.dot_general(
            pb, x1[t], (((1,), (0,)), ((), ())),
            preferred_element_type=jnp.float32).astype(bf16)


def _stage_b_kernel(y_hbm, s_ref, w3_hbm, w3q_ref, b3_ref, o_ref,
                    yv_ref, w3bf_ref, w3f0_ref, sem_y, sem_w):
    T = yv_ref.shape[1]
    HW = yv_ref.shape[3]
    Q = w3q_ref.shape[0]
    j = pl.program_id(0)
    b = pl.program_id(1)

    @pl.when(jnp.logical_and(j == 0, b == 0))
    def _():
        cpy = pltpu.make_async_copy(y_hbm, yv_ref, sem_y)
        cpy.start()
        cpw = pltpu.make_async_copy(w3_hbm.at[0:w3f0_ref.shape[0]], w3f0_ref,
                                    sem_w)
        cpw.start()
        cpw.wait()
        for t in range(T):
            sl = slice(t * HW, (t + 1) * HW)
            w3bf_ref[0, :, sl] = w3f0_ref[:, sl].astype(jnp.bfloat16)
        cpy.wait()

    def _body(cur, nxt):
        for t in range(T):
            sl = slice(t * HW, (t + 1) * HW)
            w3bf_ref[nxt, pl.ds(b * Q, Q), sl] = (
                w3q_ref[:, sl].astype(jnp.bfloat16))
        acc = s_ref[b] * b3_ref[0]
        for t in range(T):
            acc = acc + jax.lax.dot_general(
                yv_ref[b, t], w3bf_ref[cur, :, t * HW:(t + 1) * HW],
                (((1,), (1,)), ((), ())),
                preferred_element_type=jnp.float32)
        o_ref[0, 0] = acc

    @pl.when(j % 2 == 0)
    def _():
        _body(0, 1)

    @pl.when(j % 2 == 1)
    def _():
        _body(1, 0)


def kernel(input1, input2, W1, b1, W2, b2, W3, b3):
    B, T, C, H, W_ = input1.shape
    THW = T * H * W_
    HW = H * W_
    F = W1.shape[0]
    bf16 = jnp.bfloat16

    x1 = input1.astype(bf16).reshape(B, T, C, HW)
    x2 = input2.astype(bf16).reshape(B, T, C, HW)
    w1 = W1.astype(bf16).reshape(1, F, T, HW).transpose(0, 2, 1, 3)
    w2 = W2.astype(bf16).reshape(1, F, T, HW).transpose(0, 2, 1, 3)
    J = 2 * F
    w3 = W3
    b1r = b1.reshape(1, F)
    b2r = b2.reshape(1, F)
    b3r = b3.reshape(J, 1, HW // 2)

    nb = B // 2
    y, s = pl.pallas_call(
        _stage_a_kernel,
        grid=(2, nb),
        in_specs=[
            pl.BlockSpec((1, T, C, HW), lambda c, i: (c * nb + i, 0, 0, 0)),
            pl.BlockSpec((1, T, C, HW), lambda c, i: (c * nb + i, 0, 0, 0)),
            pl.BlockSpec((1, T, F, HW), lambda c, i: (0, 0, 0, 0)),
            pl.BlockSpec((1, T, F, HW), lambda c, i: (0, 0, 0, 0)),
            pl.BlockSpec((1, F), lambda c, i: (0, 0)),
            pl.BlockSpec((1, F), lambda c, i: (0, 0)),
        ],
        out_specs=[
            pl.BlockSpec((1, T, C, HW), lambda c, i: (c * nb + i, 0, 0, 0)),
            pl.BlockSpec((1, C, 1), lambda c, i: (c * nb + i, 0, 0)),
        ],
        out_shape=[
            jax.ShapeDtypeStruct((B, T, C, HW), bf16),
            jax.ShapeDtypeStruct((B, C, 1), jnp.float32),
        ],
        compiler_params=pltpu.CompilerParams(
            dimension_semantics=("arbitrary", "arbitrary"),
            allow_input_fusion=[True, True, True, True, True, True],
            vmem_limit_bytes=56 * 1024 * 1024,
        ),
        name="channel_dot_scores",
    )(x1, x2, w1, w2, b1r, b2r)

    y = pltpu.with_memory_space_constraint(y, pltpu.HBM)
    S = HW // 2
    Q = S // 4
    NQ = (F * HW) // Q
    out = pl.pallas_call(
        _stage_b_kernel,
        grid=(J, B),
        in_specs=[
            pl.BlockSpec(memory_space=pltpu.HBM),
            pl.BlockSpec(memory_space=pltpu.VMEM),
            pl.BlockSpec(memory_space=pltpu.HBM),
            pl.BlockSpec((Q, THW), lambda j, b: (((j + 1) * B + b) % NQ, 0)),
            pl.BlockSpec((1, 1, S), lambda j, b: (j, 0, 0)),
        ],
        out_specs=pl.BlockSpec(
            (1, 1, C, S), lambda j, b: (b, j // 2, 0, j % 2)),
        out_shape=jax.ShapeDtypeStruct((B, F, C, HW), jnp.float32),
        scratch_shapes=[
            pltpu.VMEM((B, T, C, HW), jnp.bfloat16),
            pltpu.VMEM((2, S, THW), jnp.bfloat16),
            pltpu.VMEM((S, THW), jnp.float32),
            pltpu.SemaphoreType.DMA,
            pltpu.SemaphoreType.DMA,
        ],
        compiler_params=pltpu.CompilerParams(
            dimension_semantics=("arbitrary", "arbitrary"),
            vmem_limit_bytes=56 * 1024 * 1024,
        ),
        name="channel_dot_apply",
    )(y, s, w3, w3, b3r)

    return out.reshape(B, F, C, H, W_)


# speedup vs baseline: 1.0280x; 1.0280x over previous
import jax
import jax.numpy as jnp
from jax.experimental import pallas as pl
from jax.experimental.pallas import tpu as pltpu


def _stage_a_kernel(x1_ref, x2_ref, w1_ref, w2_ref, b1_ref, b2_ref, y_ref, s_ref):
    T = x1_ref.shape[1]
    bf16 = jnp.bfloat16
    x1 = [x1_ref[0, t] for t in range(T)]
    x2 = [x2_ref[0, t] for t in range(T)]
    qT = b1_ref[...] + sum(
        jax.lax.dot_general(x1[t], w1_ref[0, t], (((1,), (1,)), ((), ())),
                            preferred_element_type=jnp.float32)
        for t in range(T))
    kT = b2_ref[...] + sum(
        jax.lax.dot_general(x2[t], w2_ref[0, t], (((1,), (1,)), ((), ())),
                            preferred_element_type=jnp.float32)
        for t in range(T))
    logits = jax.lax.dot_general(
        qT, kT, (((1,), (1,)), ((), ())), preferred_element_type=jnp.float32)
    m = jnp.max(logits, axis=0, keepdims=True)
    e = jnp.exp(logits - m)
    p = e / jnp.sum(e, axis=0, keepdims=True)
    s_ref[0] = jnp.sum(p, axis=1, keepdims=True)
    pb = p.astype(bf16)
    for t in range(T):
        y_ref[0, t] = jax.lax.dot_general(
            pb, x1[t], (((1,), (0,)), ((), ())),
            preferred_element_type=jnp.float32).astype(bf16)


def _stage_b_kernel(y_hbm, s_ref, w3_hbm, w3q_ref, b3_ref, o_ref,
                    yv_ref, w3bf_ref, w3f0_ref, sem_y, sem_w):
    T = yv_ref.shape[1]
    HW = yv_ref.shape[3]
    NQ = w3f0_ref.shape[0]
    HS = w3q_ref.shape[0]
    j = pl.program_id(0)
    g = pl.program_id(1)

    @pl.when(jnp.logical_and(j == 0, g == 0))
    def _():
        cpy = pltpu.make_async_copy(y_hbm, yv_ref, sem_y)
        cpy.start()
        for q in range(w3bf_ref.shape[1] // NQ):
            cpw = pltpu.make_async_copy(
                w3_hbm.at[q * NQ:(q + 1) * NQ], w3f0_ref, sem_w)
            cpw.start()
            cpw.wait()
            for t in range(T):
                sl = slice(t * HW, (t + 1) * HW)
                w3bf_ref[0, q * NQ:(q + 1) * NQ, sl] = (
                    w3f0_ref[:, sl].astype(jnp.bfloat16))
        cpy.wait()

    def _body(cur, nxt):
        for t in range(T):
            sl = slice(t * HW, (t + 1) * HW)
            w3bf_ref[nxt, pl.ds(g * HS, HS), sl] = (
                w3q_ref[:, sl].astype(jnp.bfloat16))
        for k in range(2):
            b = 2 * g + k
            acc = s_ref[b] * b3_ref[0]
            for t in range(T):
                acc = acc + jax.lax.dot_general(
                    yv_ref[b, t], w3bf_ref[cur, :, t * HW:(t + 1) * HW],
                    (((1,), (1,)), ((), ())),
                    preferred_element_type=jnp.float32)
            o_ref[k, 0] = acc

    @pl.when(j % 2 == 0)
    def _():
        _body(0, 1)

    @pl.when(j % 2 == 1)
    def _():
        _body(1, 0)


def kernel(input1, input2, W1, b1, W2, b2, W3, b3):
    B, T, C, H, W_ = input1.shape
    THW = T * H * W_
    HW = H * W_
    F = W1.shape[0]
    bf16 = jnp.bfloat16

    x1 = input1.astype(bf16).reshape(B, T, C, HW)
    x2 = input2.astype(bf16).reshape(B, T, C, HW)
    w1 = W1.astype(bf16).reshape(1, F, T, HW).transpose(0, 2, 1, 3)
    w2 = W2.astype(bf16).reshape(1, F, T, HW).transpose(0, 2, 1, 3)
    J = 2 * F
    w3 = W3
    b1r = b1.reshape(1, F)
    b2r = b2.reshape(1, F)
    b3r = b3.reshape(J, 1, HW // 2)

    nb = B // 2
    y, s = pl.pallas_call(
        _stage_a_kernel,
        grid=(2, nb),
        in_specs=[
            pl.BlockSpec((1, T, C, HW), lambda c, i: (c * nb + i, 0, 0, 0)),
            pl.BlockSpec((1, T, C, HW), lambda c, i: (c * nb + i, 0, 0, 0)),
            pl.BlockSpec((1, T, F, HW), lambda c, i: (0, 0, 0, 0)),
            pl.BlockSpec((1, T, F, HW), lambda c, i: (0, 0, 0, 0)),
            pl.BlockSpec((1, F), lambda c, i: (0, 0)),
            pl.BlockSpec((1, F), lambda c, i: (0, 0)),
        ],
        out_specs=[
            pl.BlockSpec((1, T, C, HW), lambda c, i: (c * nb + i, 0, 0, 0)),
            pl.BlockSpec((1, C, 1), lambda c, i: (c * nb + i, 0, 0)),
        ],
        out_shape=[
            jax.ShapeDtypeStruct((B, T, C, HW), bf16),
            jax.ShapeDtypeStruct((B, C, 1), jnp.float32),
        ],
        compiler_params=pltpu.CompilerParams(
            dimension_semantics=("arbitrary", "arbitrary"),
            allow_input_fusion=[True, True, True, True, True, True],
            vmem_limit_bytes=56 * 1024 * 1024,
        ),
        name="channel_dot_scores",
    )(x1, x2, w1, w2, b1r, b2r)

    y = pltpu.with_memory_space_constraint(y, pltpu.HBM)
    S = HW // 2
    Q = S // 4
    NQ = (F * HW) // Q
    NH = (F * HW) // (S // 2)
    out = pl.pallas_call(
        _stage_b_kernel,
        grid=(J, B // 2),
        in_specs=[
            pl.BlockSpec(memory_space=pltpu.HBM),
            pl.BlockSpec(memory_space=pltpu.VMEM),
            pl.BlockSpec(memory_space=pltpu.HBM),
            pl.BlockSpec((S // 2, THW),
                         lambda j, g: (((j + 1) * 2 + g) % NH, 0)),
            pl.BlockSpec((1, 1, S), lambda j, g: (j, 0, 0)),
        ],
        out_specs=pl.BlockSpec(
            (2, 1, C, S), lambda j, g: (g, j // 2, 0, j % 2)),
        out_shape=jax.ShapeDtypeStruct((B, F, C, HW), jnp.float32),
        scratch_shapes=[
            pltpu.VMEM((B, T, C, HW), jnp.bfloat16),
            pltpu.VMEM((2, S, THW), jnp.bfloat16),
            pltpu.VMEM((Q, THW), jnp.float32),
            pltpu.SemaphoreType.DMA,
            pltpu.SemaphoreType.DMA,
        ],
        compiler_params=pltpu.CompilerParams(
            dimension_semantics=("arbitrary", "arbitrary"),
            vmem_limit_bytes=56 * 1024 * 1024,
        ),
        name="channel_dot_apply",
    )(y, s, w3, w3, b3r)

    return out.reshape(B, F, C, H, W_)
